# baseline (speedup 1.0000x reference)
import functools
import math

import jax
import jax.numpy as jnp
from jax import lax
from jax.experimental import pallas as pl
from jax.experimental.pallas import tpu as pltpu
from jax.experimental.pallas import tpu_sc as plsc

D_MODEL = 64
SCALE = math.sqrt(D_MODEL)
NBUF = 2


@functools.lru_cache(maxsize=None)
def _build_lookup(b: int, s: int, d: int):
    info = plsc.get_sparse_core_info()
    nc, ns = info.num_cores, info.num_subcores
    nw = nc * ns
    assert b % nw == 0 and s % 8 == 0
    b_per_w = b // nw
    n_per_w = b_per_w * s
    d2 = 2 * d

    mesh = plsc.VectorSubcoreMesh(core_axis_name="c", subcore_axis_name="s")

    @functools.partial(
        pl.kernel,
        mesh=mesh,
        out_type=jax.ShapeDtypeStruct((b, s, d), jnp.float32),
        scratch_types=[
            pltpu.VMEM((n_per_w,), jnp.int32),
            pltpu.VMEM((n_per_w,), jnp.int32),
            pltpu.VMEM((NBUF, s, d2), jnp.float32),
            pltpu.VMEM((NBUF, s, d), jnp.float32),
            pltpu.SemaphoreType.DMA,
            pltpu.SemaphoreType.DMA,
        ],
        compiler_params=pltpu.CompilerParams(use_tc_tiling_on_sc=True),
    )
    def lookup(idx_hbm, pairs_hbm, out_hbm, idx_v, jv, buf, obuf, sem_g,
               sem_o):
        wid = lax.axis_index("s") * nc + lax.axis_index("c")
        b0 = wid * b_per_w
        pltpu.sync_copy(idx_hbm.at[pl.ds(wid * n_per_w, n_per_w)], idx_v)

        def pair_body(i, carry):
            sl = pl.ds(i * 16, 16)
            jv[sl] = lax.shift_right_logical(idx_v[sl], 1)
            return carry

        lax.fori_loop(0, n_per_w // 16, pair_body, 0)

        def gather(c, slot):
            return pltpu.make_async_copy(
                pairs_hbm.at[jv.at[pl.ds(c * s, s)]], buf.at[slot], sem_g)

        def put(c, slot):
            return pltpu.make_async_copy(
                obuf.at[slot], out_hbm.at[b0 + c], sem_o)

        for c in range(NBUF):
            gather(c, c).start()

        def chunk_body(c, carry):
            slot = lax.rem(c, NBUF)
            gather(c, slot).wait()

            @pl.when(c >= NBUF)
            def _():
                put(c - NBUF, slot).wait()

            def row_body(j16, c2):
                start = lax.min(j16 * 16, s - 16)
                bases = (idx_v[pl.ds(c * s + start, 16)] & 1) * d
                for dj in range(16):
                    j = start + dj
                    base = bases[dj]
                    for k in range(d // 16):
                        obuf[slot, j, pl.ds(k * 16, 16)] = (
                            buf[slot, j, pl.ds(base + k * 16, 16)] * SCALE)
                return c2

            lax.fori_loop(0, (s + 15) // 16, row_body, 0)

            @pl.when(c + NBUF < b_per_w)
            def _():
                gather(c + NBUF, slot).start()

            put(c, slot).start()
            return carry

        lax.fori_loop(0, b_per_w, chunk_body, 0)

        for c in range(b_per_w - NBUF, b_per_w):
            put(c, c % NBUF).wait()

    return lookup


def kernel(x, table):
    b, s = x.shape
    v, d = table.shape
    idx = x.reshape(b * s).astype(jnp.int32)
    pairs = jnp.concatenate([table[0::2], table[1::2]], axis=1)
    return _build_lookup(b, s, d)(idx, pairs)


# speedup vs baseline: 11.0672x; 11.0672x over previous
import functools
import math

import jax
import jax.numpy as jnp
from jax import lax
from jax.experimental import pallas as pl
from jax.experimental.pallas import tpu as pltpu
from jax.experimental.pallas import tpu_sc as plsc

D_MODEL = 64
SCALE = math.sqrt(D_MODEL)
NBUF = 2


@functools.lru_cache(maxsize=None)
def _build_lookup(b: int, s: int, d: int):
    info = plsc.get_sparse_core_info()
    nc, ns = info.num_cores, info.num_subcores
    nw = nc * ns
    assert b % nw == 0 and s % 8 == 0
    b_per_w = b // nw
    n_per_w = b_per_w * s
    d2 = 2 * d

    mesh = plsc.VectorSubcoreMesh(core_axis_name="c", subcore_axis_name="s")

    @functools.partial(
        pl.kernel,
        mesh=mesh,
        out_type=jax.ShapeDtypeStruct((b, s, d), jnp.float32),
        scratch_types=[
            pltpu.VMEM((n_per_w,), jnp.int32),
            pltpu.VMEM((n_per_w,), jnp.int32),
            pltpu.VMEM((NBUF, s, d2), jnp.float32),
            pltpu.VMEM((NBUF, s, d), jnp.float32),
            pltpu.SemaphoreType.DMA,
            pltpu.SemaphoreType.DMA,
        ],
        compiler_params=pltpu.CompilerParams(use_tc_tiling_on_sc=True),
    )
    def lookup(idx_hbm, pairs_hbm, out_hbm, idx_v, jv, buf, obuf, sem_g,
               sem_o):
        wid = lax.axis_index("s") * nc + lax.axis_index("c")
        b0 = wid * b_per_w
        pltpu.sync_copy(idx_hbm.at[pl.ds(wid * n_per_w, n_per_w)], idx_v)

        def pair_body(i, carry):
            sl = pl.ds(i * 16, 16)
            jv[sl] = lax.shift_right_logical(idx_v[sl], 1)
            return carry

        lax.fori_loop(0, n_per_w // 16, pair_body, 0)

        def gather(c, slot):
            return pltpu.make_async_copy(
                pairs_hbm.at[jv.at[pl.ds(c * s, s)]], buf.at[slot], sem_g)

        def put(c, slot):
            return pltpu.make_async_copy(
                obuf.at[slot], out_hbm.at[b0 + c], sem_o)

        for c in range(NBUF):
            gather(c, c).start()

        def chunk_body(c, carry):
            slot = lax.rem(c, NBUF)
            gather(c, slot).wait()

            @pl.when(c >= NBUF)
            def _():
                put(c - NBUF, slot).wait()

            def row_body(j16, c2):
                start = lax.min(j16 * 16, s - 16)
                bases = (idx_v[pl.ds(c * s + start, 16)] & 1) * d
                for dj in range(16):
                    j = start + dj
                    base = bases[dj]
                    for k in range(d // 16):
                        obuf[slot, j, pl.ds(k * 16, 16)] = (
                            buf[slot, j, pl.ds(base + k * 16, 16)] * SCALE)
                return c2

            lax.fori_loop(0, (s + 15) // 16, row_body, 0)

            @pl.when(c + NBUF < b_per_w)
            def _():
                gather(c + NBUF, slot).start()

            put(c, slot).start()
            return carry

        lax.fori_loop(0, b_per_w, chunk_body, 0)

        for c in range(b_per_w - NBUF, b_per_w):
            put(c, c % NBUF).wait()

    return lookup


def kernel(x, table):
    b, s = x.shape
    v, d = table.shape
    idx = x.reshape(b * s).astype(jnp.int32)
    pairs = table.reshape(v // 2, 2 * d)
    return _build_lookup(b, s, d)(idx, pairs)
